# Initial kernel scaffold; baseline (speedup 1.0000x reference)
import jax
import jax.numpy as jnp
from jax.experimental import pallas as pl


def kernel(x, edge_index, support_vals, emb_1, emb_2, alpha, beta):
    raise NotImplementedError("write your pallas kernel here")


# speedup vs baseline: 1.8937x; 1.8937x over previous
import functools

import jax
import jax.numpy as jnp
from jax import lax
from jax.experimental import pallas as pl
from jax.experimental.pallas import tpu as pltpu
from jax.experimental.pallas import tpu_sc as plsc

N = 10000
D = 128
E = 320000
ITERS = 3

RB = 1000
NRB = N // RB
EB = 4000
NEB = E // EB
TW = 256

NC = 2
NS = 16
NW = NC * NS
EPW = E // NW
CH = 80
NCHUNK = EPW // CH


def _proj_body(x_ref, e1_ref, e2_ref, x1_ref, x2_ref):
    xb = x_ref[...]
    x1_ref[...] = jnp.dot(xb, e1_ref[...], preferred_element_type=jnp.float32)
    x2_ref[...] = jnp.dot(xb, e2_ref[...], preferred_element_type=jnp.float32)


def _proj(x, e1, e2):
    return pl.pallas_call(
        _proj_body,
        grid=(NRB,),
        in_specs=[
            pl.BlockSpec((RB, D), lambda i: (i, 0)),
            pl.BlockSpec((D, D), lambda i: (0, 0)),
            pl.BlockSpec((D, D), lambda i: (0, 0)),
        ],
        out_specs=[
            pl.BlockSpec((RB, D), lambda i: (i, 0)),
            pl.BlockSpec((RB, D), lambda i: (i, 0)),
        ],
        out_shape=[
            jax.ShapeDtypeStruct((N, D), jnp.float32),
            jax.ShapeDtypeStruct((N, D), jnp.float32),
        ],
    )(x, e1, e2)


def _stats_body(x1_ref, x2_ref, m_ref, s_ref):
    j = pl.program_id(1)
    t = lax.dot_general(x1_ref[...], x2_ref[...],
                        (((1,), (1,)), ((), ())),
                        preferred_element_type=jnp.float32)
    t = jnp.where(t >= 0, t, 0.2 * t)
    tmax = jnp.max(t, axis=1, keepdims=True)

    @pl.when(j == 0)
    def _():
        m_ref[...] = tmax
        s_ref[...] = jnp.sum(jnp.exp(t - tmax), axis=1, keepdims=True)

    @pl.when(j > 0)
    def _():
        m_old = m_ref[...]
        s_old = s_ref[...]
        m_new = jnp.maximum(m_old, tmax)
        s_ref[...] = (s_old * jnp.exp(m_old - m_new)
                      + jnp.sum(jnp.exp(t - m_new), axis=1, keepdims=True))
        m_ref[...] = m_new


def _stats(x1, x2):
    return pl.pallas_call(
        _stats_body,
        grid=(NRB, NRB),
        in_specs=[
            pl.BlockSpec((RB, D), lambda i, j: (i, 0)),
            pl.BlockSpec((RB, D), lambda i, j: (j, 0)),
        ],
        out_specs=[
            pl.BlockSpec((RB, 1), lambda i, j: (i, 0)),
            pl.BlockSpec((RB, 1), lambda i, j: (i, 0)),
        ],
        out_shape=[
            jax.ShapeDtypeStruct((N, 1), jnp.float32),
            jax.ShapeDtypeStruct((N, 1), jnp.float32),
        ],
    )(x1, x2)


def _sc_gather(table_r, x2, rows, cols):
    mesh = plsc.VectorSubcoreMesh(core_axis_name="c", subcore_axis_name="s")

    @functools.partial(
        pl.kernel, mesh=mesh,
        out_type=[
            jax.ShapeDtypeStruct((E, TW), jnp.float32),
            jax.ShapeDtypeStruct((E, D), jnp.float32),
        ],
        scratch_types=[
            pltpu.VMEM((CH,), jnp.int32),
            pltpu.VMEM((CH,), jnp.int32),
            pltpu.VMEM((CH, TW), jnp.float32),
            pltpu.VMEM((CH, D), jnp.float32),
            pltpu.SemaphoreType.DMA,
            pltpu.SemaphoreType.DMA,
        ],
    )
    def k(tr_hbm, x2_hbm, row_hbm, col_hbm, gr_hbm, gc_hbm,
          ri, ci, gr_v, gc_v, s1, s2):
        wid = lax.axis_index("s") * NC + lax.axis_index("c")
        base = wid * EPW

        def body(i, carry):
            off = base + i * CH
            pltpu.sync_copy(row_hbm.at[pl.ds(off, CH)], ri)
            pltpu.sync_copy(col_hbm.at[pl.ds(off, CH)], ci)
            c1 = pltpu.async_copy(tr_hbm.at[ri], gr_v, s1)
            c2 = pltpu.async_copy(x2_hbm.at[ci], gc_v, s2)
            c1.wait()
            c2.wait()
            pltpu.sync_copy(gr_v, gr_hbm.at[pl.ds(off, CH)])
            pltpu.sync_copy(gc_v, gc_hbm.at[pl.ds(off, CH)])
            return carry

        lax.fori_loop(0, NCHUNK, body, 0)

    return k(table_r, x2, rows, cols)


def _vals_body(gr_ref, gc_ref, sv_ref, v_ref):
    logit = jnp.sum(gr_ref[:, :D] * gc_ref[...], axis=1, keepdims=True)
    logit = jnp.where(logit >= 0, logit, 0.2 * logit)
    m = gr_ref[:, D:D + 1]
    s = gr_ref[:, D + 1:D + 2]
    v_ref[...] = sv_ref[...] * jnp.exp(logit - m) / s


def _vals(gr, gc, sv):
    return pl.pallas_call(
        _vals_body,
        grid=(NEB,),
        in_specs=[
            pl.BlockSpec((EB, TW), lambda i: (i, 0)),
            pl.BlockSpec((EB, D), lambda i: (i, 0)),
            pl.BlockSpec((EB, 1), lambda i: (i, 0)),
        ],
        out_specs=pl.BlockSpec((EB, 1), lambda i: (i, 0)),
        out_shape=jax.ShapeDtypeStruct((E, 1), jnp.float32),
    )(gr, gc, sv)


def _sc_spmm(out_cur, cols, rows, vals, zerosD):
    mesh = plsc.VectorSubcoreMesh(core_axis_name="c", subcore_axis_name="s")

    @functools.partial(
        pl.kernel, mesh=mesh,
        out_type=jax.ShapeDtypeStruct((NC, N, D), jnp.float32),
        scratch_types=[
            pltpu.VMEM((CH,), jnp.int32),
            pltpu.VMEM((CH,), jnp.int32),
            pltpu.VMEM((CH + 16,), jnp.float32),
            pltpu.VMEM((CH, D), jnp.float32),
            pltpu.VMEM_SHARED((N, D), jnp.float32),
            pltpu.SemaphoreType.DMA,
        ],
    )
    def k(o_hbm, col_hbm, row_hbm, val_hbm, z_hbm, part_hbm,
          ci, ri, v_v, g_v, acc, sem):
        cid = lax.axis_index("c")
        sid = lax.axis_index("s")

        @pl.when(sid == 0)
        def _():
            pltpu.sync_copy(z_hbm, acc)

        plsc.subcore_barrier()
        base = (sid * NC + cid) * EPW

        def body(i, carry):
            off = base + i * CH
            pltpu.sync_copy(col_hbm.at[pl.ds(off, CH)], ci)
            pltpu.sync_copy(row_hbm.at[pl.ds(off, CH)], ri)
            pltpu.sync_copy(val_hbm.at[pl.ds(off, CH)], v_v.at[pl.ds(0, CH)])
            pltpu.async_copy(o_hbm.at[ci], g_v, sem).wait()

            def mult(e, c2):
                val = v_v[pl.ds(e, 16)][0]
                for jj in range(D // 16):
                    sl = pl.ds(jj * 16, 16)
                    g_v[e, sl] = g_v[e, sl] * val
                return c2

            lax.fori_loop(0, CH, mult, 0)
            pltpu.sync_copy(g_v, acc.at[ri], add=True)
            return carry

        lax.fori_loop(0, NCHUNK, body, 0)
        plsc.subcore_barrier()

        @pl.when(sid == 0)
        def _():
            pltpu.sync_copy(acc, part_hbm.at[cid])

    return k(out_cur, cols, rows, vals, zerosD)


def _upd_body(x_ref, o_ref, p0_ref, p1_ref, n0_ref, n1_ref,
              a_ref, b_ref, out_ref):
    a = jnp.exp(a_ref[0])
    b = jnp.exp(b_ref[0])
    norm = n0_ref[...] + n1_ref[...]
    spmm = p0_ref[...] + p1_ref[...]
    out_ref[...] = ((x_ref[...] * b + (spmm + o_ref[...]) * a)
                    / (b + norm * a + a))


def _update(x, o, p0, p1, n0, n1, a1, b1):
    return pl.pallas_call(
        _upd_body,
        grid=(NRB,),
        in_specs=[
            pl.BlockSpec((RB, D), lambda i: (i, 0)),
            pl.BlockSpec((RB, D), lambda i: (i, 0)),
            pl.BlockSpec((RB, D), lambda i: (i, 0)),
            pl.BlockSpec((RB, D), lambda i: (i, 0)),
            pl.BlockSpec((RB, D), lambda i: (i, 0)),
            pl.BlockSpec((RB, D), lambda i: (i, 0)),
            pl.BlockSpec(memory_space=pltpu.SMEM),
            pl.BlockSpec(memory_space=pltpu.SMEM),
        ],
        out_specs=pl.BlockSpec((RB, D), lambda i: (i, 0)),
        out_shape=jax.ShapeDtypeStruct((N, D), jnp.float32),
    )(x, o, p0, p1, n0, n1, a1, b1)


def kernel(x, edge_index, support_vals, emb_1, emb_2, alpha, beta):
    x = x.astype(jnp.float32)
    rows = edge_index[0]
    cols = edge_index[1]

    x1, x2 = _proj(x, emb_1, emb_2)
    m, s = _stats(x1, x2)
    table_r = jnp.concatenate(
        [x1, m, s, jnp.zeros((N, TW - D - 2), jnp.float32)], axis=1)

    gr, gc = _sc_gather(table_r, x2, rows, cols)
    sv = support_vals.reshape(E, 1)
    vals = _vals(gr, gc, sv).reshape(E)

    zerosD = jnp.zeros((N, D), jnp.float32)
    npart = _sc_spmm(jnp.ones((N, D), jnp.float32), cols, rows, vals, zerosD)
    a1 = alpha.reshape(1)
    b1 = beta.reshape(1)

    out = x
    for _ in range(ITERS):
        part = _sc_spmm(out, cols, rows, vals, zerosD)
        out = _update(x, out, part[0], part[1], npart[0], npart[1], a1, b1)
    return out


# speedup vs baseline: 2.2777x; 1.2028x over previous
import functools

import jax
import jax.numpy as jnp
from jax import lax
from jax.experimental import pallas as pl
from jax.experimental.pallas import tpu as pltpu
from jax.experimental.pallas import tpu_sc as plsc

N = 10000
D = 128
E = 320000
ITERS = 3

RB = 1000
NRB = N // RB
EB = 4000
NEB = E // EB
TW = 256

NC = 2
NS = 16
NW = NC * NS
EPW = E // NW
CH = 80
NCHUNK = EPW // CH


def _proj_body(x_ref, e1_ref, e2_ref, x1_ref, x2_ref):
    xb = x_ref[...]
    x1_ref[...] = jnp.dot(xb, e1_ref[...], preferred_element_type=jnp.float32)
    x2_ref[...] = jnp.dot(xb, e2_ref[...], preferred_element_type=jnp.float32)


def _proj(x, e1, e2):
    return pl.pallas_call(
        _proj_body,
        grid=(NRB,),
        in_specs=[
            pl.BlockSpec((RB, D), lambda i: (i, 0)),
            pl.BlockSpec((D, D), lambda i: (0, 0)),
            pl.BlockSpec((D, D), lambda i: (0, 0)),
        ],
        out_specs=[
            pl.BlockSpec((RB, D), lambda i: (i, 0)),
            pl.BlockSpec((RB, D), lambda i: (i, 0)),
        ],
        out_shape=[
            jax.ShapeDtypeStruct((N, D), jnp.float32),
            jax.ShapeDtypeStruct((N, D), jnp.float32),
        ],
    )(x, e1, e2)


def _stats_body(x1_ref, x2_ref, m_ref, s_ref):
    j = pl.program_id(1)
    t = lax.dot_general(x1_ref[...], x2_ref[...],
                        (((1,), (1,)), ((), ())),
                        preferred_element_type=jnp.float32)
    t = jnp.where(t >= 0, t, 0.2 * t)
    tmax = jnp.max(t, axis=1, keepdims=True)

    @pl.when(j == 0)
    def _():
        m_ref[...] = tmax
        s_ref[...] = jnp.sum(jnp.exp(t - tmax), axis=1, keepdims=True)

    @pl.when(j > 0)
    def _():
        m_old = m_ref[...]
        s_old = s_ref[...]
        m_new = jnp.maximum(m_old, tmax)
        s_ref[...] = (s_old * jnp.exp(m_old - m_new)
                      + jnp.sum(jnp.exp(t - m_new), axis=1, keepdims=True))
        m_ref[...] = m_new


def _stats(x1, x2):
    return pl.pallas_call(
        _stats_body,
        grid=(NRB, NRB),
        in_specs=[
            pl.BlockSpec((RB, D), lambda i, j: (i, 0)),
            pl.BlockSpec((RB, D), lambda i, j: (j, 0)),
        ],
        out_specs=[
            pl.BlockSpec((RB, 1), lambda i, j: (i, 0)),
            pl.BlockSpec((RB, 1), lambda i, j: (i, 0)),
        ],
        out_shape=[
            jax.ShapeDtypeStruct((N, 1), jnp.float32),
            jax.ShapeDtypeStruct((N, 1), jnp.float32),
        ],
    )(x1, x2)


def _sc_gather(table_r, x2, rows, cols):
    mesh = plsc.VectorSubcoreMesh(core_axis_name="c", subcore_axis_name="s")

    @functools.partial(
        pl.kernel, mesh=mesh,
        out_type=[
            jax.ShapeDtypeStruct((E, TW), jnp.float32),
            jax.ShapeDtypeStruct((E, D), jnp.float32),
        ],
        scratch_types=[
            pltpu.VMEM((CH,), jnp.int32),
            pltpu.VMEM((CH,), jnp.int32),
            pltpu.VMEM((CH, TW), jnp.float32),
            pltpu.VMEM((CH, D), jnp.float32),
            pltpu.SemaphoreType.DMA,
            pltpu.SemaphoreType.DMA,
        ],
    )
    def k(tr_hbm, x2_hbm, row_hbm, col_hbm, gr_hbm, gc_hbm,
          ri, ci, gr_v, gc_v, s1, s2):
        wid = lax.axis_index("s") * NC + lax.axis_index("c")
        base = wid * EPW

        def body(i, carry):
            off = base + i * CH
            pltpu.sync_copy(row_hbm.at[pl.ds(off, CH)], ri)
            pltpu.sync_copy(col_hbm.at[pl.ds(off, CH)], ci)
            c1 = pltpu.async_copy(tr_hbm.at[ri], gr_v, s1)
            c2 = pltpu.async_copy(x2_hbm.at[ci], gc_v, s2)
            c1.wait()
            c2.wait()
            pltpu.sync_copy(gr_v, gr_hbm.at[pl.ds(off, CH)])
            pltpu.sync_copy(gc_v, gc_hbm.at[pl.ds(off, CH)])
            return carry

        lax.fori_loop(0, NCHUNK, body, 0)

    return k(table_r, x2, rows, cols)


def _vals_body(gr_ref, gc_ref, sv_ref, v_ref):
    logit = jnp.sum(gr_ref[:, :D] * gc_ref[...], axis=1, keepdims=True)
    logit = jnp.where(logit >= 0, logit, 0.2 * logit)
    m = gr_ref[:, D:D + 1]
    s = gr_ref[:, D + 1:D + 2]
    v_ref[...] = sv_ref[...] * jnp.exp(logit - m) / s


def _vals(gr, gc, sv):
    return pl.pallas_call(
        _vals_body,
        grid=(NEB,),
        in_specs=[
            pl.BlockSpec((EB, TW), lambda i: (i, 0)),
            pl.BlockSpec((EB, D), lambda i: (i, 0)),
            pl.BlockSpec((EB, 1), lambda i: (i, 0)),
        ],
        out_specs=pl.BlockSpec((EB, 1), lambda i: (i, 0)),
        out_shape=jax.ShapeDtypeStruct((E, 1), jnp.float32),
    )(gr, gc, sv)


def _sc_spmm(out_cur, cols, rows, vals, zerosD):
    mesh = plsc.VectorSubcoreMesh(core_axis_name="c", subcore_axis_name="s")

    @functools.partial(
        pl.kernel, mesh=mesh,
        out_type=jax.ShapeDtypeStruct((NC, N, D), jnp.float32),
        scratch_types=[
            pltpu.VMEM((CH,), jnp.int32),
            pltpu.VMEM((CH,), jnp.int32),
            pltpu.VMEM((CH,), jnp.int32),
            pltpu.VMEM((CH + 16,), jnp.float32),
            pltpu.VMEM((CH, D), jnp.float32),
            pltpu.VMEM((CH, D), jnp.float32),
            pltpu.VMEM_SHARED((N, D), jnp.float32),
            pltpu.SemaphoreType.DMA,
            pltpu.SemaphoreType.DMA,
        ],
    )
    def k(o_hbm, col_hbm, row_hbm, val_hbm, z_hbm, part_hbm,
          ci0, ci1, ri, v_v, g_v0, g_v1, acc, sem0, sem1):
        cid = lax.axis_index("c")
        sid = lax.axis_index("s")

        @pl.when(sid == 0)
        def _():
            pltpu.sync_copy(z_hbm, acc)

        base = (sid * NC + cid) * EPW

        def start(i, ci, g_v, sem):
            pltpu.sync_copy(col_hbm.at[pl.ds(base + i * CH, CH)], ci)
            pltpu.async_copy(o_hbm.at[ci], g_v, sem)

        def finish(i, ci, g_v, sem, ci_n, g_n, sem_n):
            @pl.when(i + 1 < NCHUNK)
            def _():
                start(i + 1, ci_n, g_n, sem_n)

            off = base + i * CH
            pltpu.sync_copy(row_hbm.at[pl.ds(off, CH)], ri)
            pltpu.sync_copy(val_hbm.at[pl.ds(off, CH)], v_v.at[pl.ds(0, CH)])
            pltpu.make_async_copy(o_hbm.at[ci], g_v, sem).wait()

            def mult(e, c2):
                val = v_v[pl.ds(e, 16)][0]
                for jj in range(D // 16):
                    sl = pl.ds(jj * 16, 16)
                    g_v[e, sl] = g_v[e, sl] * val
                return c2

            lax.fori_loop(0, CH, mult, 0)
            pltpu.sync_copy(g_v, acc.at[ri], add=True)

        start(0, ci0, g_v0, sem0)
        plsc.subcore_barrier()

        def body(i, carry):
            @pl.when(lax.rem(i, 2) == 0)
            def _():
                finish(i, ci0, g_v0, sem0, ci1, g_v1, sem1)

            @pl.when(lax.rem(i, 2) == 1)
            def _():
                finish(i, ci1, g_v1, sem1, ci0, g_v0, sem0)

            return carry

        lax.fori_loop(0, NCHUNK, body, 0)
        plsc.subcore_barrier()

        @pl.when(sid == 0)
        def _():
            pltpu.sync_copy(acc, part_hbm.at[cid])

    return k(out_cur, cols, rows, vals, zerosD)


def _upd_body(x_ref, o_ref, p0_ref, p1_ref, n0_ref, n1_ref,
              a_ref, b_ref, out_ref):
    a = jnp.exp(a_ref[0])
    b = jnp.exp(b_ref[0])
    norm = n0_ref[...] + n1_ref[...]
    spmm = p0_ref[...] + p1_ref[...]
    out_ref[...] = ((x_ref[...] * b + (spmm + o_ref[...]) * a)
                    / (b + norm * a + a))


def _update(x, o, p0, p1, n0, n1, a1, b1):
    return pl.pallas_call(
        _upd_body,
        grid=(NRB,),
        in_specs=[
            pl.BlockSpec((RB, D), lambda i: (i, 0)),
            pl.BlockSpec((RB, D), lambda i: (i, 0)),
            pl.BlockSpec((RB, D), lambda i: (i, 0)),
            pl.BlockSpec((RB, D), lambda i: (i, 0)),
            pl.BlockSpec((RB, D), lambda i: (i, 0)),
            pl.BlockSpec((RB, D), lambda i: (i, 0)),
            pl.BlockSpec(memory_space=pltpu.SMEM),
            pl.BlockSpec(memory_space=pltpu.SMEM),
        ],
        out_specs=pl.BlockSpec((RB, D), lambda i: (i, 0)),
        out_shape=jax.ShapeDtypeStruct((N, D), jnp.float32),
    )(x, o, p0, p1, n0, n1, a1, b1)


def kernel(x, edge_index, support_vals, emb_1, emb_2, alpha, beta):
    x = x.astype(jnp.float32)
    rows = edge_index[0]
    cols = edge_index[1]

    x1, x2 = _proj(x, emb_1, emb_2)
    m, s = _stats(x1, x2)
    table_r = jnp.concatenate(
        [x1, m, s, jnp.zeros((N, TW - D - 2), jnp.float32)], axis=1)

    gr, gc = _sc_gather(table_r, x2, rows, cols)
    sv = support_vals.reshape(E, 1)
    vals = _vals(gr, gc, sv).reshape(E)

    zerosD = jnp.zeros((N, D), jnp.float32)
    npart = _sc_spmm(jnp.ones((N, D), jnp.float32), cols, rows, vals, zerosD)
    a1 = alpha.reshape(1)
    b1 = beta.reshape(1)

    out = x
    for _ in range(ITERS):
        part = _sc_spmm(out, cols, rows, vals, zerosD)
        out = _update(x, out, part[0], part[1], npart[0], npart[1], a1, b1)
    return out


# speedup vs baseline: 2.3962x; 1.0520x over previous
import functools

import jax
import jax.numpy as jnp
from jax import lax
from jax.experimental import pallas as pl
from jax.experimental.pallas import tpu as pltpu
from jax.experimental.pallas import tpu_sc as plsc

N = 10000
D = 128
E = 320000
ITERS = 3

RB = 1000
NRB = N // RB
EB = 4000
NEB = E // EB
TW = 256

NC = 2
NS = 16
NW = NC * NS
EPW = E // NW
CH = 80
NCHUNK = EPW // CH


def _proj_body(x_ref, e1_ref, e2_ref, x1_ref, x2_ref):
    xb = x_ref[...]
    x1_ref[...] = jnp.dot(xb, e1_ref[...], preferred_element_type=jnp.float32)
    x2_ref[...] = jnp.dot(xb, e2_ref[...], preferred_element_type=jnp.float32)


def _proj(x, e1, e2):
    return pl.pallas_call(
        _proj_body,
        grid=(NRB,),
        in_specs=[
            pl.BlockSpec((RB, D), lambda i: (i, 0)),
            pl.BlockSpec((D, D), lambda i: (0, 0)),
            pl.BlockSpec((D, D), lambda i: (0, 0)),
        ],
        out_specs=[
            pl.BlockSpec((RB, D), lambda i: (i, 0)),
            pl.BlockSpec((RB, D), lambda i: (i, 0)),
        ],
        out_shape=[
            jax.ShapeDtypeStruct((N, D), jnp.float32),
            jax.ShapeDtypeStruct((N, D), jnp.float32),
        ],
    )(x, e1, e2)


def _stats_body(x1_ref, x2_ref, m_ref, s_ref):
    j = pl.program_id(1)
    t = lax.dot_general(x1_ref[...], x2_ref[...],
                        (((1,), (1,)), ((), ())),
                        preferred_element_type=jnp.float32)
    t = jnp.where(t >= 0, t, 0.2 * t)
    tmax = jnp.max(t, axis=1, keepdims=True)

    @pl.when(j == 0)
    def _():
        m_ref[...] = tmax
        s_ref[...] = jnp.sum(jnp.exp(t - tmax), axis=1, keepdims=True)

    @pl.when(j > 0)
    def _():
        m_old = m_ref[...]
        s_old = s_ref[...]
        m_new = jnp.maximum(m_old, tmax)
        s_ref[...] = (s_old * jnp.exp(m_old - m_new)
                      + jnp.sum(jnp.exp(t - m_new), axis=1, keepdims=True))
        m_ref[...] = m_new


def _stats(x1, x2):
    return pl.pallas_call(
        _stats_body,
        grid=(NRB, NRB),
        in_specs=[
            pl.BlockSpec((RB, D), lambda i, j: (i, 0)),
            pl.BlockSpec((RB, D), lambda i, j: (j, 0)),
        ],
        out_specs=[
            pl.BlockSpec((RB, 1), lambda i, j: (i, 0)),
            pl.BlockSpec((RB, 1), lambda i, j: (i, 0)),
        ],
        out_shape=[
            jax.ShapeDtypeStruct((N, 1), jnp.float32),
            jax.ShapeDtypeStruct((N, 1), jnp.float32),
        ],
    )(x1, x2)


def _sc_gather(table_r, x2, rows, cols):
    mesh = plsc.VectorSubcoreMesh(core_axis_name="c", subcore_axis_name="s")

    @functools.partial(
        pl.kernel, mesh=mesh,
        out_type=[
            jax.ShapeDtypeStruct((E, TW), jnp.float32),
            jax.ShapeDtypeStruct((E, D), jnp.float32),
        ],
        scratch_types=[
            pltpu.VMEM((CH,), jnp.int32),
            pltpu.VMEM((CH,), jnp.int32),
            pltpu.VMEM((CH,), jnp.int32),
            pltpu.VMEM((CH,), jnp.int32),
            pltpu.VMEM((CH, TW), jnp.float32),
            pltpu.VMEM((CH, TW), jnp.float32),
            pltpu.VMEM((CH, D), jnp.float32),
            pltpu.VMEM((CH, D), jnp.float32),
            pltpu.SemaphoreType.DMA,
            pltpu.SemaphoreType.DMA,
            pltpu.SemaphoreType.DMA,
            pltpu.SemaphoreType.DMA,
        ],
    )
    def k(tr_hbm, x2_hbm, row_hbm, col_hbm, gr_hbm, gc_hbm,
          ri0, ri1, ci0, ci1, gr_v0, gr_v1, gc_v0, gc_v1, sa0, sa1, sb0, sb1):
        wid = lax.axis_index("s") * NC + lax.axis_index("c")
        base = wid * EPW

        def start(i, ri, ci, gr_v, gc_v, sa, sb):
            off = base + i * CH
            pltpu.sync_copy(row_hbm.at[pl.ds(off, CH)], ri)
            pltpu.sync_copy(col_hbm.at[pl.ds(off, CH)], ci)
            pltpu.async_copy(tr_hbm.at[ri], gr_v, sa)
            pltpu.async_copy(x2_hbm.at[ci], gc_v, sb)

        def finish(i, ri, ci, gr_v, gc_v, sa, sb,
                   ri_n, ci_n, gr_n, gc_n, sa_n, sb_n):
            @pl.when(i + 1 < NCHUNK)
            def _():
                start(i + 1, ri_n, ci_n, gr_n, gc_n, sa_n, sb_n)

            off = base + i * CH
            pltpu.make_async_copy(tr_hbm.at[ri], gr_v, sa).wait()
            pltpu.make_async_copy(x2_hbm.at[ci], gc_v, sb).wait()
            pltpu.sync_copy(gr_v, gr_hbm.at[pl.ds(off, CH)])
            pltpu.sync_copy(gc_v, gc_hbm.at[pl.ds(off, CH)])

        start(0, ri0, ci0, gr_v0, gc_v0, sa0, sb0)

        def body(i, carry):
            @pl.when(lax.rem(i, 2) == 0)
            def _():
                finish(i, ri0, ci0, gr_v0, gc_v0, sa0, sb0,
                       ri1, ci1, gr_v1, gc_v1, sa1, sb1)

            @pl.when(lax.rem(i, 2) == 1)
            def _():
                finish(i, ri1, ci1, gr_v1, gc_v1, sa1, sb1,
                       ri0, ci0, gr_v0, gc_v0, sa0, sb0)

            return carry

        lax.fori_loop(0, NCHUNK, body, 0)

    return k(table_r, x2, rows, cols)


def _vals_body(gr_ref, gc_ref, sv_ref, v_ref):
    logit = jnp.sum(gr_ref[:, :D] * gc_ref[...], axis=1, keepdims=True)
    logit = jnp.where(logit >= 0, logit, 0.2 * logit)
    m = gr_ref[:, D:D + 1]
    s = gr_ref[:, D + 1:D + 2]
    v_ref[...] = sv_ref[...] * jnp.exp(logit - m) / s


def _vals(gr, gc, sv):
    return pl.pallas_call(
        _vals_body,
        grid=(NEB,),
        in_specs=[
            pl.BlockSpec((EB, TW), lambda i: (i, 0)),
            pl.BlockSpec((EB, D), lambda i: (i, 0)),
            pl.BlockSpec((EB, 1), lambda i: (i, 0)),
        ],
        out_specs=pl.BlockSpec((EB, 1), lambda i: (i, 0)),
        out_shape=jax.ShapeDtypeStruct((E, 1), jnp.float32),
    )(gr, gc, sv)


def _sc_spmm(out_cur, cols, rows, vals, zerosD):
    mesh = plsc.VectorSubcoreMesh(core_axis_name="c", subcore_axis_name="s")

    @functools.partial(
        pl.kernel, mesh=mesh,
        out_type=jax.ShapeDtypeStruct((NC, N, D), jnp.float32),
        scratch_types=[
            pltpu.VMEM((CH,), jnp.int32),
            pltpu.VMEM((CH,), jnp.int32),
            pltpu.VMEM((CH,), jnp.int32),
            pltpu.VMEM((CH + 16,), jnp.float32),
            pltpu.VMEM((CH, D), jnp.float32),
            pltpu.VMEM((CH, D), jnp.float32),
            pltpu.VMEM_SHARED((N, D), jnp.float32),
            pltpu.SemaphoreType.DMA,
            pltpu.SemaphoreType.DMA,
        ],
    )
    def k(o_hbm, col_hbm, row_hbm, val_hbm, z_hbm, part_hbm,
          ci0, ci1, ri, v_v, g_v0, g_v1, acc, sem0, sem1):
        cid = lax.axis_index("c")
        sid = lax.axis_index("s")

        @pl.when(sid == 0)
        def _():
            pltpu.sync_copy(z_hbm, acc)

        base = (sid * NC + cid) * EPW

        def start(i, ci, g_v, sem):
            pltpu.sync_copy(col_hbm.at[pl.ds(base + i * CH, CH)], ci)
            pltpu.async_copy(o_hbm.at[ci], g_v, sem)

        def finish(i, ci, g_v, sem, ci_n, g_n, sem_n):
            @pl.when(i + 1 < NCHUNK)
            def _():
                start(i + 1, ci_n, g_n, sem_n)

            off = base + i * CH
            pltpu.sync_copy(row_hbm.at[pl.ds(off, CH)], ri)
            pltpu.sync_copy(val_hbm.at[pl.ds(off, CH)], v_v.at[pl.ds(0, CH)])
            pltpu.make_async_copy(o_hbm.at[ci], g_v, sem).wait()

            def mult(e, c2):
                val = v_v[pl.ds(e, 16)][0]
                for jj in range(D // 16):
                    sl = pl.ds(jj * 16, 16)
                    g_v[e, sl] = g_v[e, sl] * val
                return c2

            lax.fori_loop(0, CH, mult, 0)
            pltpu.sync_copy(g_v, acc.at[ri], add=True)

        start(0, ci0, g_v0, sem0)
        plsc.subcore_barrier()

        def body(i, carry):
            @pl.when(lax.rem(i, 2) == 0)
            def _():
                finish(i, ci0, g_v0, sem0, ci1, g_v1, sem1)

            @pl.when(lax.rem(i, 2) == 1)
            def _():
                finish(i, ci1, g_v1, sem1, ci0, g_v0, sem0)

            return carry

        lax.fori_loop(0, NCHUNK, body, 0)
        plsc.subcore_barrier()

        @pl.when(sid == 0)
        def _():
            pltpu.sync_copy(acc, part_hbm.at[cid])

    return k(out_cur, cols, rows, vals, zerosD)


def _upd_body(x_ref, o_ref, p0_ref, p1_ref, n0_ref, n1_ref,
              a_ref, b_ref, out_ref):
    a = jnp.exp(a_ref[0])
    b = jnp.exp(b_ref[0])
    norm = n0_ref[...] + n1_ref[...]
    spmm = p0_ref[...] + p1_ref[...]
    out_ref[...] = ((x_ref[...] * b + (spmm + o_ref[...]) * a)
                    / (b + norm * a + a))


def _update(x, o, p0, p1, n0, n1, a1, b1):
    return pl.pallas_call(
        _upd_body,
        grid=(NRB,),
        in_specs=[
            pl.BlockSpec((RB, D), lambda i: (i, 0)),
            pl.BlockSpec((RB, D), lambda i: (i, 0)),
            pl.BlockSpec((RB, D), lambda i: (i, 0)),
            pl.BlockSpec((RB, D), lambda i: (i, 0)),
            pl.BlockSpec((RB, D), lambda i: (i, 0)),
            pl.BlockSpec((RB, D), lambda i: (i, 0)),
            pl.BlockSpec(memory_space=pltpu.SMEM),
            pl.BlockSpec(memory_space=pltpu.SMEM),
        ],
        out_specs=pl.BlockSpec((RB, D), lambda i: (i, 0)),
        out_shape=jax.ShapeDtypeStruct((N, D), jnp.float32),
    )(x, o, p0, p1, n0, n1, a1, b1)


def kernel(x, edge_index, support_vals, emb_1, emb_2, alpha, beta):
    x = x.astype(jnp.float32)
    rows = edge_index[0]
    cols = edge_index[1]

    x1, x2 = _proj(x, emb_1, emb_2)
    m, s = _stats(x1, x2)
    table_r = jnp.concatenate(
        [x1, m, s, jnp.zeros((N, TW - D - 2), jnp.float32)], axis=1)

    gr, gc = _sc_gather(table_r, x2, rows, cols)
    sv = support_vals.reshape(E, 1)
    vals = _vals(gr, gc, sv).reshape(E)

    zerosD = jnp.zeros((N, D), jnp.float32)
    npart = _sc_spmm(jnp.ones((N, D), jnp.float32), cols, rows, vals, zerosD)
    a1 = alpha.reshape(1)
    b1 = beta.reshape(1)

    out = x
    for _ in range(ITERS):
        part = _sc_spmm(out, cols, rows, vals, zerosD)
        out = _update(x, out, part[0], part[1], npart[0], npart[1], a1, b1)
    return out
